# Initial kernel scaffold; baseline (speedup 1.0000x reference)
import jax
import jax.numpy as jnp
from jax.experimental import pallas as pl


def kernel(node_feats, edge_index, graph_ids, W1, b1, Wr1, br1, g1, be1, W2, b2, Wr2, br2, g2, be2, Wspec, bspec, Wsh, bsh, Wg, bgate, Wf1, bf1, gf1, bef1, Wf2, bf2, gf2, bef2, Wo, bo):
    raise NotImplementedError("write your pallas kernel here")


# speedup vs baseline: 5.9484x; 5.9484x over previous
import functools

import jax
import jax.numpy as jnp
from jax import lax
from jax.experimental import pallas as pl
from jax.experimental.pallas import tpu as pltpu
from jax.experimental.pallas import tpu_sc as plsc

_F32 = jnp.float32
_NC = 2
_NS = 16
_N_GRAPHS = 256


def _mm(a, b):
    return lax.dot_general(
        a, b, (((a.ndim - 1,), (0,)), ((), ())),
        precision=lax.Precision.HIGHEST, preferred_element_type=_F32)


def _bn(x, gamma, beta, eps=1e-5):
    mu = jnp.mean(x, axis=0)
    var = jnp.mean((x - mu[None, :]) ** 2, axis=0)
    return gamma[None, :] * (x - mu[None, :]) / jnp.sqrt(var + eps)[None, :] + beta[None, :]


def _sc_degree_hist(src2, dst2, n_nodes):
    nw, e_t = src2.shape
    mesh = plsc.VectorSubcoreMesh(core_axis_name="c", subcore_axis_name="s")

    @functools.partial(
        pl.kernel, mesh=mesh,
        out_type=(jax.ShapeDtypeStruct((nw, n_nodes), _F32),
                  jax.ShapeDtypeStruct((nw, n_nodes), _F32)),
        scratch_types=[pltpu.VMEM((e_t,), jnp.int32),
                       pltpu.VMEM((e_t,), jnp.int32),
                       pltpu.VMEM((n_nodes,), _F32),
                       pltpu.VMEM((n_nodes,), _F32)],
        compiler_params=pltpu.CompilerParams(needs_layout_passes=False),
    )
    def deg_kernel(src_hbm, dst_hbm, out_o, out_i, src_v, dst_v, ho_v, hi_v):
        c = lax.axis_index("c")
        s = lax.axis_index("s")
        wid = s * _NC + c
        pltpu.sync_copy(src_hbm.at[wid], src_v)
        pltpu.sync_copy(dst_hbm.at[wid], dst_v)
        zeros = jnp.zeros((16,), _F32)

        @pl.loop(0, n_nodes // 16)
        def _zero(i):
            ho_v[pl.ds(i * 16, 16)] = zeros
            hi_v[pl.ds(i * 16, 16)] = zeros

        ones = jnp.ones((16,), _F32)

        @pl.loop(0, e_t // 16)
        def _hist(i):
            plsc.addupdate_scatter(ho_v, [src_v[pl.ds(i * 16, 16)]], ones)
            plsc.addupdate_scatter(hi_v, [dst_v[pl.ds(i * 16, 16)]], ones)

        pltpu.sync_copy(ho_v, out_o.at[wid])
        pltpu.sync_copy(hi_v, out_i.at[wid])

    return deg_kernel(src2, dst2)


def _sc_edge_aggregate(h_tab, src4, dst3, n_pad):
    nc, ns, nch, kk = src4.shape
    d2 = h_tab.shape[1]
    rpt = n_pad // ns
    zr = 128
    zn = rpt // zr
    mesh = plsc.VectorSubcoreMesh(core_axis_name="c", subcore_axis_name="s")

    @functools.partial(
        pl.kernel, mesh=mesh,
        out_type=jax.ShapeDtypeStruct((nc, n_pad, d2), _F32),
        scratch_types=[pltpu.VMEM((nch, kk), jnp.int32),
                       pltpu.VMEM((nch, kk), jnp.int32),
                       pltpu.VMEM((kk, d2), _F32),
                       pltpu.VMEM((zr, d2), _F32),
                       pltpu.VMEM_SHARED((n_pad, d2), _F32),
                       pltpu.SemaphoreType.DMA],
        compiler_params=pltpu.CompilerParams(needs_layout_passes=False,
                                             use_tc_tiling_on_sc=False),
    )
    def agg_kernel(h_hbm, src_hbm, dst_hbm, out_hbm, src_v, dst_v, gbuf, zbuf,
                   agg_sh, sem):
        c = lax.axis_index("c")
        s = lax.axis_index("s")
        pltpu.sync_copy(src_hbm.at[c, s], src_v)
        pltpu.sync_copy(dst_hbm.at[s], dst_v)
        zeros = jnp.zeros((16,), _F32)

        @pl.loop(0, zr)
        def _zero(i):
            @pl.loop(0, d2 // 16)
            def _zero_row(j):
                zbuf[i, pl.ds(j * 16, 16)] = zeros

        @pl.loop(0, zn)
        def _zinit(t):
            pltpu.sync_copy(zbuf, agg_sh.at[pl.ds(s * rpt + t * zr, zr)])

        plsc.subcore_barrier()

        @pl.loop(0, nch)
        def _edges(j):
            pltpu.async_copy(h_hbm.at[src_v.at[j]], gbuf, sem).wait()
            pltpu.sync_copy(gbuf, agg_sh.at[dst_v.at[j]], add=True)

        plsc.subcore_barrier()
        pltpu.sync_copy(agg_sh.at[pl.ds(s * rpt, rpt)],
                        out_hbm.at[c, pl.ds(s * rpt, rpt)])

    return agg_kernel(h_tab, src4, dst3)


def _tc_pre(x, hist_o, hist_i, W1, Wr1, br1):
    n, d_in = x.shape
    d_hid = W1.shape[1]
    d2 = d_hid // 2

    def body(x_ref, ho_ref, hi_ref, w1_ref, wr1_ref, br1_ref,
             h_out, r_out, ro_out, ri_out):
        xv = x_ref[...]
        dego = jnp.maximum(jnp.sum(ho_ref[...], axis=0), 1.0)
        degi = jnp.maximum(jnp.sum(hi_ref[...], axis=0), 1.0)
        rsd_o = lax.rsqrt(dego)
        rsd_i = lax.rsqrt(degi)
        h = _mm(xv, w1_ref[...]) * rsd_o[:, None]
        h_out[...] = jnp.concatenate([h[:, :d2], h[:, d2:]], axis=0)
        r_out[...] = jax.nn.relu(_mm(xv, wr1_ref[...]) + br1_ref[...][None, :])
        ro_out[...] = rsd_o
        ri_out[...] = rsd_i

    return pl.pallas_call(
        body,
        out_shape=(jax.ShapeDtypeStruct((2 * n, d2), _F32),
                   jax.ShapeDtypeStruct((n, d_hid), _F32),
                   jax.ShapeDtypeStruct((n,), _F32),
                   jax.ShapeDtypeStruct((n,), _F32)),
    )(x, hist_o, hist_i, W1, Wr1, br1)


def _tc_mid(p, r1, rsd_i, rsd_o, b1, g1, be1, W2, Wr2, br2):
    n, d_hid = r1.shape
    d_out = W2.shape[1]
    d2 = d_out // 2

    def body(p_ref, r1_ref, ri_ref, ro_ref, b1_ref, g1_ref, be1_ref,
             w2_ref, wr2_ref, br2_ref, h2_out, r2_out):
        agg = jnp.concatenate([p_ref[0][:n], p_ref[1][:n]], axis=1)
        pre = jax.nn.relu(agg * ri_ref[...][:, None] + b1_ref[...][None, :]) + r1_ref[...]
        h1 = _bn(pre, g1_ref[...], be1_ref[...])
        h2 = _mm(h1, w2_ref[...]) * ro_ref[...][:, None]
        h2_out[...] = jnp.concatenate([h2[:, :d2], h2[:, d2:]], axis=0)
        r2_out[...] = jax.nn.relu(_mm(h1, wr2_ref[...]) + br2_ref[...][None, :])

    return pl.pallas_call(
        body,
        out_shape=(jax.ShapeDtypeStruct((2 * n, d2), _F32),
                   jax.ShapeDtypeStruct((n, d_out), _F32)),
    )(p, r1, rsd_i, rsd_o, b1, g1, be1, W2, Wr2, br2)


def _tc_final(p2, r2, rsd_i, b2, g2, be2, gids, wspec_t, bspec_v,
              Wg, bgate, Wf1, bf1, gf1, bef1, Wf2, bf2, gf2, bef2, Wo, bo,
              n_graphs, n_tasks):
    n, d_out = r2.shape

    def body(p_ref, r2_ref, ri_ref, b2_ref, g2_ref, be2_ref, gid_ref,
             wspec_ref, bspec_ref, wg_ref, bgate_ref, wf1_ref, bf1_ref,
             gf1_ref, bef1_ref, wf2_ref, bf2_ref, gf2_ref, bef2_ref,
             wo_ref, bo_ref, out_ref):
        agg = jnp.concatenate([p_ref[0][:n], p_ref[1][:n]], axis=1)
        pre = jax.nn.relu(agg * ri_ref[...][:, None] + b2_ref[...][None, :]) + r2_ref[...]
        h2 = _bn(pre, g2_ref[...], be2_ref[...])

        w = jax.nn.sigmoid(_mm(h2, wspec_ref[...]) + bspec_ref[...][None, :])
        ids = gid_ref[...]
        onehot = (ids[:, None] ==
                  lax.broadcasted_iota(jnp.int32, (n, n_graphs), 1)).astype(_F32)
        xcat = jnp.concatenate(
            [h2 * w[:, i][:, None] for i in range(n_tasks)] + [h2], axis=1)
        seg = lax.dot_general(onehot, xcat, (((0,), (0,)), ((), ())),
                              precision=lax.Precision.HIGHEST,
                              preferred_element_type=_F32)
        counts = jnp.maximum(jnp.sum(onehot, axis=0), 1.0)
        feats = [seg[:, i * d_out:(i + 1) * d_out] for i in range(n_tasks)]
        hg = seg[:, n_tasks * d_out:(n_tasks + 1) * d_out] / counts[:, None]
        prim = feats[n_tasks - 1]

        gc = jnp.zeros((n_graphs, d_out), _F32)
        for i in range(n_tasks - 1):
            logits = _mm(hg, wg_ref[i]) + bgate_ref[i][None, :]
            gate = jax.nn.softmax(logits, axis=-1)
            gc = gc + feats[i] * gate[:, 0][:, None] + prim * gate[:, 1][:, None]

        combine2 = [feats[0], gc, feats[1], feats[2], feats[3]]
        preds = []
        for i in range(n_tasks):
            a = jax.nn.relu(_mm(combine2[i], wf1_ref[i]) + bf1_ref[i][None, :])
            a = _bn(a, gf1_ref[i], bef1_ref[i])
            a = jax.nn.relu(_mm(a, wf2_ref[i]) + bf2_ref[i][None, :])
            a = _bn(a, gf2_ref[i], bef2_ref[i])
            preds.append(_mm(a, wo_ref[i]) + bo_ref[i][None, :])
        out_ref[...] = jnp.concatenate(preds, axis=1)

    return pl.pallas_call(
        body,
        out_shape=jax.ShapeDtypeStruct((n_graphs, n_tasks), _F32),
    )(p2, r2, rsd_i, b2, g2, be2, gids, wspec_t, bspec_v,
      Wg, bgate, Wf1, bf1, gf1, bef1, Wf2, bf2, gf2, bef2, Wo, bo)


def kernel(node_feats, edge_index, graph_ids, W1, b1, Wr1, br1, g1, be1,
           W2, b2, Wr2, br2, g2, be2, Wspec, bspec, Wsh, bsh, Wg, bgate,
           Wf1, bf1, gf1, bef1, Wf2, bf2, gf2, bef2, Wo, bo):
    n = node_feats.shape[0]
    e = edge_index.shape[1]
    n_tasks = Wspec.shape[0]

    src = edge_index[0].astype(jnp.int32)
    dst = edge_index[1].astype(jnp.int32)
    nw = _NC * _NS
    src2 = src.reshape(nw, e // nw)
    dst2 = dst.reshape(nw, e // nw)

    kk = 80
    nch = e // (_NS * kk)
    src4 = jnp.stack([src, src + n]).reshape(_NC, _NS, nch, kk)
    dst3 = dst.reshape(_NS, nch, kk)

    quantum = 128 * _NS
    n_pad = ((n + quantum - 1) // quantum) * quantum
    hist_o, hist_i = _sc_degree_hist(src2, dst2, n)
    h1in, r1, rsd_o, rsd_i = _tc_pre(node_feats, hist_o, hist_i, W1, Wr1, br1)
    p1 = _sc_edge_aggregate(h1in, src4, dst3, n_pad)
    h2in, r2 = _tc_mid(p1, r1, rsd_i, rsd_o, b1, g1, be1, W2, Wr2, br2)
    p2 = _sc_edge_aggregate(h2in, src4, dst3, n_pad)

    wspec_t = jnp.transpose(Wspec[:, :, 0])
    bspec_v = bspec[:, 0]
    return _tc_final(p2, r2, rsd_i, b2, g2, be2, graph_ids.astype(jnp.int32),
                     wspec_t, bspec_v, Wg, bgate, Wf1, bf1, gf1, bef1,
                     Wf2, bf2, gf2, bef2, Wo, bo, _N_GRAPHS, n_tasks)


# speedup vs baseline: 10.4489x; 1.7566x over previous
import functools

import jax
import jax.numpy as jnp
from jax import lax
from jax.experimental import pallas as pl
from jax.experimental.pallas import tpu as pltpu
from jax.experimental.pallas import tpu_sc as plsc

_F32 = jnp.float32
_NC = 2
_NS = 16
_N_GRAPHS = 256
_NB = 5


def _mm(a, b):
    return lax.dot_general(
        a, b, (((a.ndim - 1,), (0,)), ((), ())),
        precision=lax.Precision.HIGHEST, preferred_element_type=_F32)


def _bn(x, gamma, beta, eps=1e-5):
    mu = jnp.mean(x, axis=0)
    var = jnp.mean((x - mu[None, :]) ** 2, axis=0)
    return gamma[None, :] * (x - mu[None, :]) / jnp.sqrt(var + eps)[None, :] + beta[None, :]


def _sc_degree_hist(src2, dst2, n_nodes):
    nw, e_t = src2.shape
    mesh = plsc.VectorSubcoreMesh(core_axis_name="c", subcore_axis_name="s")

    @functools.partial(
        pl.kernel, mesh=mesh,
        out_type=(jax.ShapeDtypeStruct((nw, n_nodes), _F32),
                  jax.ShapeDtypeStruct((nw, n_nodes), _F32)),
        scratch_types=[pltpu.VMEM((e_t,), jnp.int32),
                       pltpu.VMEM((e_t,), jnp.int32),
                       pltpu.VMEM((n_nodes,), _F32),
                       pltpu.VMEM((n_nodes,), _F32)],
        compiler_params=pltpu.CompilerParams(needs_layout_passes=False),
    )
    def deg_kernel(src_hbm, dst_hbm, out_o, out_i, src_v, dst_v, ho_v, hi_v):
        c = lax.axis_index("c")
        s = lax.axis_index("s")
        wid = s * _NC + c
        pltpu.sync_copy(src_hbm.at[wid], src_v)
        pltpu.sync_copy(dst_hbm.at[wid], dst_v)
        zeros = jnp.zeros((16,), _F32)

        @pl.loop(0, n_nodes // 16)
        def _zero(i):
            ho_v[pl.ds(i * 16, 16)] = zeros
            hi_v[pl.ds(i * 16, 16)] = zeros

        ones = jnp.ones((16,), _F32)

        @pl.loop(0, e_t // 16)
        def _hist(i):
            plsc.addupdate_scatter(ho_v, [src_v[pl.ds(i * 16, 16)]], ones)
            plsc.addupdate_scatter(hi_v, [dst_v[pl.ds(i * 16, 16)]], ones)

        pltpu.sync_copy(ho_v, out_o.at[wid])
        pltpu.sync_copy(hi_v, out_i.at[wid])

    return deg_kernel(src2, dst2)


def _sc_edge_aggregate(h_nd, src3, dst3, n_pad, kk, nb):
    nc, ns, epc = src3.shape
    nch = epc // kk
    src4 = src3.reshape(nc, ns, nch, kk)
    dst4 = dst3.reshape(nc, ns, nch, kk)
    d = h_nd.shape[1]
    rpt = n_pad // ns
    zn = rpt // kk
    assert rpt % kk == 0 and nch % nb == 0
    mesh = plsc.VectorSubcoreMesh(core_axis_name="c", subcore_axis_name="s")

    @functools.partial(
        pl.kernel, mesh=mesh,
        out_type=jax.ShapeDtypeStruct((nc, n_pad, d), _F32),
        scratch_types=[pltpu.VMEM((nch, kk), jnp.int32),
                       pltpu.VMEM((nch, kk), jnp.int32)]
                      + [pltpu.VMEM((kk, d), _F32) for _ in range(nb)]
                      + [pltpu.VMEM_SHARED((n_pad, d), _F32)]
                      + [pltpu.SemaphoreType.DMA for _ in range(nb)],
        compiler_params=pltpu.CompilerParams(needs_layout_passes=False,
                                             use_tc_tiling_on_sc=False),
    )
    def agg_kernel(h_hbm, src_hbm, dst_hbm, out_hbm, src_v, dst_v, *rest):
        gbufs = rest[:nb]
        agg_sh = rest[nb]
        sems = rest[nb + 1:]
        c = lax.axis_index("c")
        s = lax.axis_index("s")
        pltpu.sync_copy(src_hbm.at[c, s], src_v)
        pltpu.sync_copy(dst_hbm.at[c, s], dst_v)
        zeros = jnp.zeros((16,), _F32)

        @pl.loop(0, kk)
        def _zero(i):
            @pl.loop(0, d // 16)
            def _zero_row(j):
                gbufs[0][i, pl.ds(j * 16, 16)] = zeros

        @pl.loop(0, zn)
        def _zinit(t):
            pltpu.sync_copy(gbufs[0], agg_sh.at[pl.ds(s * rpt + t * kk, kk)])

        plsc.subcore_barrier()

        @pl.loop(0, nch // nb)
        def _rounds(g):
            j0 = g * nb
            cps = [pltpu.async_copy(h_hbm.at[src_v.at[j0 + b]], gbufs[b],
                                    sems[b]) for b in range(nb)]
            for b in range(nb):
                cps[b].wait()
                pltpu.sync_copy(gbufs[b], agg_sh.at[dst_v.at[j0 + b]],
                                add=True)

        plsc.subcore_barrier()
        pltpu.sync_copy(agg_sh.at[pl.ds(s * rpt, rpt)],
                        out_hbm.at[c, pl.ds(s * rpt, rpt)])

    return agg_kernel(h_nd, src4, dst4)


def _tc_pre(x, hist_o, hist_i, W1, Wr1, br1):
    n, d_in = x.shape
    d_hid = W1.shape[1]

    def body(x_ref, ho_ref, hi_ref, w1_ref, wr1_ref, br1_ref,
             h_out, r_out, ro_out, ri_out):
        xv = x_ref[...]
        dego = jnp.maximum(jnp.sum(ho_ref[...], axis=0), 1.0)
        degi = jnp.maximum(jnp.sum(hi_ref[...], axis=0), 1.0)
        rsd_o = lax.rsqrt(dego)
        rsd_i = lax.rsqrt(degi)
        h_out[...] = _mm(xv, w1_ref[...]) * rsd_o[:, None]
        r_out[...] = jax.nn.relu(_mm(xv, wr1_ref[...]) + br1_ref[...][None, :])
        ro_out[...] = rsd_o
        ri_out[...] = rsd_i

    return pl.pallas_call(
        body,
        out_shape=(jax.ShapeDtypeStruct((n, d_hid), _F32),
                   jax.ShapeDtypeStruct((n, d_hid), _F32),
                   jax.ShapeDtypeStruct((n,), _F32),
                   jax.ShapeDtypeStruct((n,), _F32)),
    )(x, hist_o, hist_i, W1, Wr1, br1)


def _tc_mid(p, r1, rsd_i, rsd_o, b1, g1, be1, W2, Wr2, br2):
    n, d_hid = r1.shape
    d_out = W2.shape[1]

    def body(p_ref, r1_ref, ri_ref, ro_ref, b1_ref, g1_ref, be1_ref,
             w2_ref, wr2_ref, br2_ref, h2_out, r2_out):
        agg = p_ref[0][:n] + p_ref[1][:n]
        pre = jax.nn.relu(agg * ri_ref[...][:, None] + b1_ref[...][None, :]) + r1_ref[...]
        h1 = _bn(pre, g1_ref[...], be1_ref[...])
        h2_out[...] = _mm(h1, w2_ref[...]) * ro_ref[...][:, None]
        r2_out[...] = jax.nn.relu(_mm(h1, wr2_ref[...]) + br2_ref[...][None, :])

    return pl.pallas_call(
        body,
        out_shape=(jax.ShapeDtypeStruct((n, d_out), _F32),
                   jax.ShapeDtypeStruct((n, d_out), _F32)),
    )(p, r1, rsd_i, rsd_o, b1, g1, be1, W2, Wr2, br2)


def _tc_final(p2, r2, rsd_i, b2, g2, be2, gids, wspec_t, bspec_v,
              Wg, bgate, Wf1, bf1, gf1, bef1, Wf2, bf2, gf2, bef2, Wo, bo,
              n_graphs, n_tasks):
    n, d_out = r2.shape

    def body(p_ref, r2_ref, ri_ref, b2_ref, g2_ref, be2_ref, gid_ref,
             wspec_ref, bspec_ref, wg_ref, bgate_ref, wf1_ref, bf1_ref,
             gf1_ref, bef1_ref, wf2_ref, bf2_ref, gf2_ref, bef2_ref,
             wo_ref, bo_ref, out_ref):
        agg = p_ref[0][:n] + p_ref[1][:n]
        pre = jax.nn.relu(agg * ri_ref[...][:, None] + b2_ref[...][None, :]) + r2_ref[...]
        h2 = _bn(pre, g2_ref[...], be2_ref[...])

        w = jax.nn.sigmoid(_mm(h2, wspec_ref[...]) + bspec_ref[...][None, :])
        ids = gid_ref[...]
        onehot = (ids[:, None] ==
                  lax.broadcasted_iota(jnp.int32, (n, n_graphs), 1)).astype(_F32)
        xcat = jnp.concatenate(
            [h2 * w[:, i][:, None] for i in range(n_tasks)] + [h2], axis=1)
        seg = lax.dot_general(onehot, xcat, (((0,), (0,)), ((), ())),
                              precision=lax.Precision.HIGHEST,
                              preferred_element_type=_F32)
        counts = jnp.maximum(jnp.sum(onehot, axis=0), 1.0)
        feats = [seg[:, i * d_out:(i + 1) * d_out] for i in range(n_tasks)]
        hg = seg[:, n_tasks * d_out:(n_tasks + 1) * d_out] / counts[:, None]
        prim = feats[n_tasks - 1]

        gc = jnp.zeros((n_graphs, d_out), _F32)
        for i in range(n_tasks - 1):
            logits = _mm(hg, wg_ref[i]) + bgate_ref[i][None, :]
            gate = jax.nn.softmax(logits, axis=-1)
            gc = gc + feats[i] * gate[:, 0][:, None] + prim * gate[:, 1][:, None]

        combine2 = [feats[0], gc, feats[1], feats[2], feats[3]]
        preds = []
        for i in range(n_tasks):
            a = jax.nn.relu(_mm(combine2[i], wf1_ref[i]) + bf1_ref[i][None, :])
            a = _bn(a, gf1_ref[i], bef1_ref[i])
            a = jax.nn.relu(_mm(a, wf2_ref[i]) + bf2_ref[i][None, :])
            a = _bn(a, gf2_ref[i], bef2_ref[i])
            preds.append(_mm(a, wo_ref[i]) + bo_ref[i][None, :])
        out_ref[...] = jnp.concatenate(preds, axis=1)

    return pl.pallas_call(
        body,
        out_shape=jax.ShapeDtypeStruct((n_graphs, n_tasks), _F32),
    )(p2, r2, rsd_i, b2, g2, be2, gids, wspec_t, bspec_v,
      Wg, bgate, Wf1, bf1, gf1, bef1, Wf2, bf2, gf2, bef2, Wo, bo)


def kernel(node_feats, edge_index, graph_ids, W1, b1, Wr1, br1, g1, be1,
           W2, b2, Wr2, br2, g2, be2, Wspec, bspec, Wsh, bsh, Wg, bgate,
           Wf1, bf1, gf1, bef1, Wf2, bf2, gf2, bef2, Wo, bo):
    n = node_feats.shape[0]
    e = edge_index.shape[1]
    n_tasks = Wspec.shape[0]

    src = edge_index[0].astype(jnp.int32)
    dst = edge_index[1].astype(jnp.int32)
    nw = _NC * _NS
    src2 = src.reshape(nw, e // nw)
    dst2 = dst.reshape(nw, e // nw)

    src3 = src.reshape(_NC, _NS, e // nw)
    dst3 = dst.reshape(_NC, _NS, e // nw)

    quantum = 80 * _NS * 2
    n_pad = ((n + quantum - 1) // quantum) * quantum
    hist_o, hist_i = _sc_degree_hist(src2, dst2, n)
    h1, r1, rsd_o, rsd_i = _tc_pre(node_feats, hist_o, hist_i, W1, Wr1, br1)
    p1 = _sc_edge_aggregate(h1, src3, dst3, n_pad, kk=40, nb=5)
    h2, r2 = _tc_mid(p1, r1, rsd_i, rsd_o, b1, g1, be1, W2, Wr2, br2)
    p2 = _sc_edge_aggregate(h2, src3, dst3, n_pad, kk=80, nb=5)

    wspec_t = jnp.transpose(Wspec[:, :, 0])
    bspec_v = bspec[:, 0]
    return _tc_final(p2, r2, rsd_i, b2, g2, be2, graph_ids.astype(jnp.int32),
                     wspec_t, bspec_v, Wg, bgate, Wf1, bf1, gf1, bef1,
                     Wf2, bf2, gf2, bef2, Wo, bo, _N_GRAPHS, n_tasks)


# speedup vs baseline: 10.7807x; 1.0318x over previous
import functools

import jax
import jax.numpy as jnp
from jax import lax
from jax.experimental import pallas as pl
from jax.experimental.pallas import tpu as pltpu
from jax.experimental.pallas import tpu_sc as plsc

_F32 = jnp.float32
_NC = 2
_NS = 16
_N_GRAPHS = 256
_NB = 5


def _mm(a, b):
    return lax.dot_general(
        a, b, (((a.ndim - 1,), (0,)), ((), ())),
        precision=lax.Precision.HIGHEST, preferred_element_type=_F32)


def _bn(x, gamma, beta, eps=1e-5):
    mu = jnp.mean(x, axis=0)
    var = jnp.mean((x - mu[None, :]) ** 2, axis=0)
    return gamma[None, :] * (x - mu[None, :]) / jnp.sqrt(var + eps)[None, :] + beta[None, :]


def _sc_degree_hist(src2, dst2, n_nodes):
    nw, e_t = src2.shape
    mesh = plsc.VectorSubcoreMesh(core_axis_name="c", subcore_axis_name="s")

    @functools.partial(
        pl.kernel, mesh=mesh,
        out_type=(jax.ShapeDtypeStruct((nw, n_nodes), _F32),
                  jax.ShapeDtypeStruct((nw, n_nodes), _F32)),
        scratch_types=[pltpu.VMEM((e_t,), jnp.int32),
                       pltpu.VMEM((e_t,), jnp.int32),
                       pltpu.VMEM((n_nodes,), _F32),
                       pltpu.VMEM((n_nodes,), _F32)],
        compiler_params=pltpu.CompilerParams(needs_layout_passes=False),
    )
    def deg_kernel(src_hbm, dst_hbm, out_o, out_i, src_v, dst_v, ho_v, hi_v):
        c = lax.axis_index("c")
        s = lax.axis_index("s")
        wid = s * _NC + c
        pltpu.sync_copy(src_hbm.at[wid], src_v)
        pltpu.sync_copy(dst_hbm.at[wid], dst_v)
        zeros = jnp.zeros((16,), _F32)

        @pl.loop(0, n_nodes // 16)
        def _zero(i):
            ho_v[pl.ds(i * 16, 16)] = zeros
            hi_v[pl.ds(i * 16, 16)] = zeros

        ones = jnp.ones((16,), _F32)

        @pl.loop(0, e_t // 16)
        def _hist(i):
            plsc.addupdate_scatter(ho_v, [src_v[pl.ds(i * 16, 16)]], ones)
            plsc.addupdate_scatter(hi_v, [dst_v[pl.ds(i * 16, 16)]], ones)

        pltpu.sync_copy(ho_v, out_o.at[wid])
        pltpu.sync_copy(hi_v, out_i.at[wid])

    return deg_kernel(src2, dst2)


def _sc_edge_aggregate(h_nd, src3, dst3, n_pad, kk, nb):
    nc, ns, epc = src3.shape
    nch = epc // kk
    src4 = src3.reshape(nc, ns, nch, kk)
    dst4 = dst3.reshape(nc, ns, nch, kk)
    d = h_nd.shape[1]
    rpt = n_pad // ns
    zn = rpt // kk
    assert rpt % kk == 0 and nch % nb == 0
    mesh = plsc.VectorSubcoreMesh(core_axis_name="c", subcore_axis_name="s")

    @functools.partial(
        pl.kernel, mesh=mesh,
        out_type=jax.ShapeDtypeStruct((nc, n_pad, d), _F32),
        scratch_types=[pltpu.VMEM((nch, kk), jnp.int32),
                       pltpu.VMEM((nch, kk), jnp.int32)]
                      + [pltpu.VMEM((kk, d), _F32) for _ in range(nb)]
                      + [pltpu.VMEM_SHARED((n_pad, d), _F32)]
                      + [pltpu.SemaphoreType.DMA for _ in range(nb)],
        compiler_params=pltpu.CompilerParams(needs_layout_passes=False,
                                             use_tc_tiling_on_sc=False),
    )
    def agg_kernel(h_hbm, src_hbm, dst_hbm, out_hbm, src_v, dst_v, *rest):
        gbufs = rest[:nb]
        agg_sh = rest[nb]
        sems = rest[nb + 1:]
        c = lax.axis_index("c")
        s = lax.axis_index("s")
        pltpu.sync_copy(src_hbm.at[c, s], src_v)
        pltpu.sync_copy(dst_hbm.at[c, s], dst_v)
        zeros = jnp.zeros((16,), _F32)

        @pl.loop(0, kk)
        def _zero(i):
            @pl.loop(0, d // 16)
            def _zero_row(j):
                gbufs[0][i, pl.ds(j * 16, 16)] = zeros

        @pl.loop(0, zn)
        def _zinit(t):
            pltpu.sync_copy(gbufs[0], agg_sh.at[pl.ds(s * rpt + t * kk, kk)])

        plsc.subcore_barrier()

        @pl.loop(0, nch // nb)
        def _rounds(g):
            j0 = g * nb
            cps = [pltpu.async_copy(h_hbm.at[src_v.at[j0 + b]], gbufs[b],
                                    sems[b]) for b in range(nb)]
            for b in range(nb):
                cps[b].wait()
                pltpu.sync_copy(gbufs[b], agg_sh.at[dst_v.at[j0 + b]],
                                add=True)

        plsc.subcore_barrier()
        pltpu.sync_copy(agg_sh.at[pl.ds(s * rpt, rpt)],
                        out_hbm.at[c, pl.ds(s * rpt, rpt)])

    return agg_kernel(h_nd, src4, dst4)


def _tc_mats(x, W1, Wr1, br1):
    n, d_in = x.shape
    d_hid = W1.shape[1]

    def body(x_ref, w1_ref, wr1_ref, br1_ref, xw_out, r_out):
        xv = x_ref[...]
        xw_out[...] = _mm(xv, w1_ref[...])
        r_out[...] = jax.nn.relu(_mm(xv, wr1_ref[...]) + br1_ref[...][None, :])

    return pl.pallas_call(
        body,
        out_shape=(jax.ShapeDtypeStruct((n, d_hid), _F32),
                   jax.ShapeDtypeStruct((n, d_hid), _F32)),
    )(x, W1, Wr1, br1)


def _tc_scale(xw, hist_o, hist_i):
    n, d_hid = xw.shape

    def body(xw_ref, ho_ref, hi_ref, h_out, ro_out, ri_out):
        dego = jnp.maximum(jnp.sum(ho_ref[...], axis=0), 1.0)
        degi = jnp.maximum(jnp.sum(hi_ref[...], axis=0), 1.0)
        rsd_o = lax.rsqrt(dego)
        h_out[...] = xw_ref[...] * rsd_o[:, None]
        ro_out[...] = rsd_o
        ri_out[...] = lax.rsqrt(degi)

    return pl.pallas_call(
        body,
        out_shape=(jax.ShapeDtypeStruct((n, d_hid), _F32),
                   jax.ShapeDtypeStruct((n,), _F32),
                   jax.ShapeDtypeStruct((n,), _F32)),
    )(xw, hist_o, hist_i)


def _tc_mid_a(p, r1, rsd_i, rsd_o, b1, g1, be1, W2):
    n, d_hid = r1.shape
    d_out = W2.shape[1]

    def body(p_ref, r1_ref, ri_ref, ro_ref, b1_ref, g1_ref, be1_ref,
             w2_ref, h2_out, h1_out):
        agg = p_ref[0][:n] + p_ref[1][:n]
        pre = jax.nn.relu(agg * ri_ref[...][:, None] + b1_ref[...][None, :]) + r1_ref[...]
        h1 = _bn(pre, g1_ref[...], be1_ref[...])
        h2_out[...] = _mm(h1, w2_ref[...]) * ro_ref[...][:, None]
        h1_out[...] = h1

    return pl.pallas_call(
        body,
        out_shape=(jax.ShapeDtypeStruct((n, d_out), _F32),
                   jax.ShapeDtypeStruct((n, d_hid), _F32)),
    )(p, r1, rsd_i, rsd_o, b1, g1, be1, W2)


def _tc_mid_b(h1, Wr2, br2):
    n = h1.shape[0]
    d_out = Wr2.shape[1]

    def body(h1_ref, wr2_ref, br2_ref, r2_out):
        r2_out[...] = jax.nn.relu(_mm(h1_ref[...], wr2_ref[...]) + br2_ref[...][None, :])

    return pl.pallas_call(
        body,
        out_shape=jax.ShapeDtypeStruct((n, d_out), _F32),
    )(h1, Wr2, br2)


def _tc_final(p2, r2, rsd_i, b2, g2, be2, gids, wspec_t, bspec_v,
              Wg, bgate, Wf1, bf1, gf1, bef1, Wf2, bf2, gf2, bef2, Wo, bo,
              n_graphs, n_tasks):
    n, d_out = r2.shape

    def body(p_ref, r2_ref, ri_ref, b2_ref, g2_ref, be2_ref, gid_ref,
             wspec_ref, bspec_ref, wg_ref, bgate_ref, wf1_ref, bf1_ref,
             gf1_ref, bef1_ref, wf2_ref, bf2_ref, gf2_ref, bef2_ref,
             wo_ref, bo_ref, out_ref):
        agg = p_ref[0][:n] + p_ref[1][:n]
        pre = jax.nn.relu(agg * ri_ref[...][:, None] + b2_ref[...][None, :]) + r2_ref[...]
        h2 = _bn(pre, g2_ref[...], be2_ref[...])

        w = jax.nn.sigmoid(_mm(h2, wspec_ref[...]) + bspec_ref[...][None, :])
        ids = gid_ref[...]
        onehot = (ids[:, None] ==
                  lax.broadcasted_iota(jnp.int32, (n, n_graphs), 1)).astype(_F32)
        xcat = jnp.concatenate(
            [h2 * w[:, i][:, None] for i in range(n_tasks)] + [h2], axis=1)
        seg = lax.dot_general(onehot, xcat, (((0,), (0,)), ((), ())),
                              precision=lax.Precision.HIGHEST,
                              preferred_element_type=_F32)
        counts = jnp.maximum(jnp.sum(onehot, axis=0), 1.0)
        feats = [seg[:, i * d_out:(i + 1) * d_out] for i in range(n_tasks)]
        hg = seg[:, n_tasks * d_out:(n_tasks + 1) * d_out] / counts[:, None]
        prim = feats[n_tasks - 1]

        gc = jnp.zeros((n_graphs, d_out), _F32)
        for i in range(n_tasks - 1):
            logits = _mm(hg, wg_ref[i]) + bgate_ref[i][None, :]
            gate = jax.nn.softmax(logits, axis=-1)
            gc = gc + feats[i] * gate[:, 0][:, None] + prim * gate[:, 1][:, None]

        combine2 = [feats[0], gc, feats[1], feats[2], feats[3]]
        preds = []
        for i in range(n_tasks):
            a = jax.nn.relu(_mm(combine2[i], wf1_ref[i]) + bf1_ref[i][None, :])
            a = _bn(a, gf1_ref[i], bef1_ref[i])
            a = jax.nn.relu(_mm(a, wf2_ref[i]) + bf2_ref[i][None, :])
            a = _bn(a, gf2_ref[i], bef2_ref[i])
            preds.append(_mm(a, wo_ref[i]) + bo_ref[i][None, :])
        out_ref[...] = jnp.concatenate(preds, axis=1)

    return pl.pallas_call(
        body,
        out_shape=jax.ShapeDtypeStruct((n_graphs, n_tasks), _F32),
    )(p2, r2, rsd_i, b2, g2, be2, gids, wspec_t, bspec_v,
      Wg, bgate, Wf1, bf1, gf1, bef1, Wf2, bf2, gf2, bef2, Wo, bo)


def kernel(node_feats, edge_index, graph_ids, W1, b1, Wr1, br1, g1, be1,
           W2, b2, Wr2, br2, g2, be2, Wspec, bspec, Wsh, bsh, Wg, bgate,
           Wf1, bf1, gf1, bef1, Wf2, bf2, gf2, bef2, Wo, bo):
    n = node_feats.shape[0]
    e = edge_index.shape[1]
    n_tasks = Wspec.shape[0]

    src = edge_index[0].astype(jnp.int32)
    dst = edge_index[1].astype(jnp.int32)
    nw = _NC * _NS
    src2 = src.reshape(nw, e // nw)
    dst2 = dst.reshape(nw, e // nw)

    src3 = src.reshape(_NC, _NS, e // nw)
    dst3 = dst.reshape(_NC, _NS, e // nw)

    quantum = 80 * _NS * 2
    n_pad = ((n + quantum - 1) // quantum) * quantum
    hist_o, hist_i = _sc_degree_hist(src2, dst2, n)
    xw, r1 = _tc_mats(node_feats, W1, Wr1, br1)
    h1, rsd_o, rsd_i = _tc_scale(xw, hist_o, hist_i)
    p1 = _sc_edge_aggregate(h1, src3, dst3, n_pad, kk=40, nb=5)
    h2, h1bn = _tc_mid_a(p1, r1, rsd_i, rsd_o, b1, g1, be1, W2)
    r2 = _tc_mid_b(h1bn, Wr2, br2)
    p2 = _sc_edge_aggregate(h2, src3, dst3, n_pad, kk=80, nb=5)

    wspec_t = jnp.transpose(Wspec[:, :, 0])
    bspec_v = bspec[:, 0]
    return _tc_final(p2, r2, rsd_i, b2, g2, be2, graph_ids.astype(jnp.int32),
                     wspec_t, bspec_v, Wg, bgate, Wf1, bf1, gf1, bef1,
                     Wf2, bf2, gf2, bef2, Wo, bo, _N_GRAPHS, n_tasks)


# speedup vs baseline: 11.3497x; 1.0528x over previous
import functools

import jax
import jax.numpy as jnp
from jax import lax
from jax.experimental import pallas as pl
from jax.experimental.pallas import tpu as pltpu
from jax.experimental.pallas import tpu_sc as plsc

_F32 = jnp.float32
_NC = 2
_NS = 16
_N_GRAPHS = 256
_NB = 5


def _mm(a, b):
    return lax.dot_general(
        a, b, (((a.ndim - 1,), (0,)), ((), ())),
        precision=lax.Precision.HIGHEST, preferred_element_type=_F32)


def _bn(x, gamma, beta, eps=1e-5):
    mu = jnp.mean(x, axis=0)
    var = jnp.mean((x - mu[None, :]) ** 2, axis=0)
    return gamma[None, :] * (x - mu[None, :]) / jnp.sqrt(var + eps)[None, :] + beta[None, :]


def _sc_degree_hist(src2, dst2, n_nodes):
    nw, e_t = src2.shape
    mesh = plsc.VectorSubcoreMesh(core_axis_name="c", subcore_axis_name="s")

    @functools.partial(
        pl.kernel, mesh=mesh,
        out_type=(jax.ShapeDtypeStruct((nw, n_nodes), _F32),
                  jax.ShapeDtypeStruct((nw, n_nodes), _F32)),
        scratch_types=[pltpu.VMEM((e_t,), jnp.int32),
                       pltpu.VMEM((e_t,), jnp.int32),
                       pltpu.VMEM((n_nodes,), _F32),
                       pltpu.VMEM((n_nodes,), _F32)],
        compiler_params=pltpu.CompilerParams(needs_layout_passes=False),
    )
    def deg_kernel(src_hbm, dst_hbm, out_o, out_i, src_v, dst_v, ho_v, hi_v):
        c = lax.axis_index("c")
        s = lax.axis_index("s")
        wid = s * _NC + c
        pltpu.sync_copy(src_hbm.at[wid], src_v)
        pltpu.sync_copy(dst_hbm.at[wid], dst_v)
        zeros = jnp.zeros((16,), _F32)

        @pl.loop(0, n_nodes // 16)
        def _zero(i):
            ho_v[pl.ds(i * 16, 16)] = zeros
            hi_v[pl.ds(i * 16, 16)] = zeros

        ones = jnp.ones((16,), _F32)

        @pl.loop(0, e_t // 16)
        def _hist(i):
            plsc.addupdate_scatter(ho_v, [src_v[pl.ds(i * 16, 16)]], ones)
            plsc.addupdate_scatter(hi_v, [dst_v[pl.ds(i * 16, 16)]], ones)

        pltpu.sync_copy(ho_v, out_o.at[wid])
        pltpu.sync_copy(hi_v, out_i.at[wid])

    return deg_kernel(src2, dst2)


def _sc_edge_aggregate(h_nd, src3, dst3, n_pad, kk, nb):
    nc, ns, epc = src3.shape
    nch = epc // kk
    src4 = src3.reshape(nc, ns, nch, kk)
    dst4 = dst3.reshape(nc, ns, nch, kk)
    d = h_nd.shape[1]
    rpt = n_pad // ns
    zn = rpt // kk
    assert rpt % kk == 0 and nch % nb == 0
    mesh = plsc.VectorSubcoreMesh(core_axis_name="c", subcore_axis_name="s")

    @functools.partial(
        pl.kernel, mesh=mesh,
        out_type=jax.ShapeDtypeStruct((nc, n_pad, d), _F32),
        scratch_types=[pltpu.VMEM((nch, kk), jnp.int32),
                       pltpu.VMEM((nch, kk), jnp.int32)]
                      + [pltpu.VMEM((kk, d), _F32) for _ in range(nb)]
                      + [pltpu.VMEM_SHARED((n_pad, d), _F32)]
                      + [pltpu.SemaphoreType.DMA for _ in range(2 * nb)],
        compiler_params=pltpu.CompilerParams(needs_layout_passes=False,
                                             use_tc_tiling_on_sc=False),
    )
    def agg_kernel(h_hbm, src_hbm, dst_hbm, out_hbm, src_v, dst_v, *rest):
        gbufs = rest[:nb]
        agg_sh = rest[nb]
        sems = rest[nb + 1:nb + 1 + nb]
        ssems = rest[nb + 1 + nb:]
        c = lax.axis_index("c")
        s = lax.axis_index("s")
        pltpu.sync_copy(src_hbm.at[c, s], src_v)
        pltpu.sync_copy(dst_hbm.at[c, s], dst_v)
        zeros = jnp.zeros((16,), _F32)

        @pl.loop(0, kk)
        def _zero(i):
            @pl.loop(0, d // 16)
            def _zero_row(j):
                gbufs[0][i, pl.ds(j * 16, 16)] = zeros

        @pl.loop(0, zn)
        def _zinit(t):
            pltpu.sync_copy(gbufs[0], agg_sh.at[pl.ds(s * rpt + t * kk, kk)])

        plsc.subcore_barrier()

        @pl.loop(0, nch // nb)
        def _rounds(g):
            j0 = g * nb
            cps = [pltpu.async_copy(h_hbm.at[src_v.at[j0 + b]], gbufs[b],
                                    sems[b]) for b in range(nb)]
            scs = []
            for b in range(nb):
                cps[b].wait()
                scs.append(pltpu.async_copy(gbufs[b],
                                            agg_sh.at[dst_v.at[j0 + b]],
                                            ssems[b], add=True))
            for b in range(nb):
                scs[b].wait()

        plsc.subcore_barrier()
        pltpu.sync_copy(agg_sh.at[pl.ds(s * rpt, rpt)],
                        out_hbm.at[c, pl.ds(s * rpt, rpt)])

    return agg_kernel(h_nd, src4, dst4)


def _tc_mats(x, W1, Wr1, br1):
    n, d_in = x.shape
    d_hid = W1.shape[1]

    def body(x_ref, w1_ref, wr1_ref, br1_ref, xw_out, r_out):
        xv = x_ref[...]
        xw_out[...] = _mm(xv, w1_ref[...])
        r_out[...] = jax.nn.relu(_mm(xv, wr1_ref[...]) + br1_ref[...][None, :])

    return pl.pallas_call(
        body,
        out_shape=(jax.ShapeDtypeStruct((n, d_hid), _F32),
                   jax.ShapeDtypeStruct((n, d_hid), _F32)),
    )(x, W1, Wr1, br1)


def _tc_scale(xw, hist_o, hist_i):
    n, d_hid = xw.shape

    def body(xw_ref, ho_ref, hi_ref, h_out, ro_out, ri_out):
        dego = jnp.maximum(jnp.sum(ho_ref[...], axis=0), 1.0)
        degi = jnp.maximum(jnp.sum(hi_ref[...], axis=0), 1.0)
        rsd_o = lax.rsqrt(dego)
        h_out[...] = xw_ref[...] * rsd_o[:, None]
        ro_out[...] = rsd_o
        ri_out[...] = lax.rsqrt(degi)

    return pl.pallas_call(
        body,
        out_shape=(jax.ShapeDtypeStruct((n, d_hid), _F32),
                   jax.ShapeDtypeStruct((n,), _F32),
                   jax.ShapeDtypeStruct((n,), _F32)),
    )(xw, hist_o, hist_i)


def _tc_mid_a(p, r1, rsd_i, rsd_o, b1, g1, be1, W2):
    n, d_hid = r1.shape
    d_out = W2.shape[1]

    def body(p_ref, r1_ref, ri_ref, ro_ref, b1_ref, g1_ref, be1_ref,
             w2_ref, h2_out, h1_out):
        agg = p_ref[0][:n] + p_ref[1][:n]
        pre = jax.nn.relu(agg * ri_ref[...][:, None] + b1_ref[...][None, :]) + r1_ref[...]
        h1 = _bn(pre, g1_ref[...], be1_ref[...])
        h2_out[...] = _mm(h1, w2_ref[...]) * ro_ref[...][:, None]
        h1_out[...] = h1

    return pl.pallas_call(
        body,
        out_shape=(jax.ShapeDtypeStruct((n, d_out), _F32),
                   jax.ShapeDtypeStruct((n, d_hid), _F32)),
    )(p, r1, rsd_i, rsd_o, b1, g1, be1, W2)


def _tc_mid_b(h1, Wr2, br2):
    n = h1.shape[0]
    d_out = Wr2.shape[1]

    def body(h1_ref, wr2_ref, br2_ref, r2_out):
        r2_out[...] = jax.nn.relu(_mm(h1_ref[...], wr2_ref[...]) + br2_ref[...][None, :])

    return pl.pallas_call(
        body,
        out_shape=jax.ShapeDtypeStruct((n, d_out), _F32),
    )(h1, Wr2, br2)


def _tc_final(p2, r2, rsd_i, b2, g2, be2, gids, wspec_t, bspec_v,
              Wg, bgate, Wf1, bf1, gf1, bef1, Wf2, bf2, gf2, bef2, Wo, bo,
              n_graphs, n_tasks):
    n, d_out = r2.shape

    def body(p_ref, r2_ref, ri_ref, b2_ref, g2_ref, be2_ref, gid_ref,
             wspec_ref, bspec_ref, wg_ref, bgate_ref, wf1_ref, bf1_ref,
             gf1_ref, bef1_ref, wf2_ref, bf2_ref, gf2_ref, bef2_ref,
             wo_ref, bo_ref, out_ref):
        agg = p_ref[0][:n] + p_ref[1][:n]
        pre = jax.nn.relu(agg * ri_ref[...][:, None] + b2_ref[...][None, :]) + r2_ref[...]
        h2 = _bn(pre, g2_ref[...], be2_ref[...])

        w = jax.nn.sigmoid(_mm(h2, wspec_ref[...]) + bspec_ref[...][None, :])
        ids = gid_ref[...]
        onehot = (ids[:, None] ==
                  lax.broadcasted_iota(jnp.int32, (n, n_graphs), 1)).astype(_F32)
        xcat = jnp.concatenate(
            [h2 * w[:, i][:, None] for i in range(n_tasks)] + [h2], axis=1)
        seg = lax.dot_general(onehot, xcat, (((0,), (0,)), ((), ())),
                              precision=lax.Precision.HIGHEST,
                              preferred_element_type=_F32)
        counts = jnp.maximum(jnp.sum(onehot, axis=0), 1.0)
        feats = [seg[:, i * d_out:(i + 1) * d_out] for i in range(n_tasks)]
        hg = seg[:, n_tasks * d_out:(n_tasks + 1) * d_out] / counts[:, None]
        prim = feats[n_tasks - 1]

        gc = jnp.zeros((n_graphs, d_out), _F32)
        for i in range(n_tasks - 1):
            logits = _mm(hg, wg_ref[i]) + bgate_ref[i][None, :]
            gate = jax.nn.softmax(logits, axis=-1)
            gc = gc + feats[i] * gate[:, 0][:, None] + prim * gate[:, 1][:, None]

        combine2 = [feats[0], gc, feats[1], feats[2], feats[3]]
        preds = []
        for i in range(n_tasks):
            a = jax.nn.relu(_mm(combine2[i], wf1_ref[i]) + bf1_ref[i][None, :])
            a = _bn(a, gf1_ref[i], bef1_ref[i])
            a = jax.nn.relu(_mm(a, wf2_ref[i]) + bf2_ref[i][None, :])
            a = _bn(a, gf2_ref[i], bef2_ref[i])
            preds.append(_mm(a, wo_ref[i]) + bo_ref[i][None, :])
        out_ref[...] = jnp.concatenate(preds, axis=1)

    return pl.pallas_call(
        body,
        out_shape=jax.ShapeDtypeStruct((n_graphs, n_tasks), _F32),
    )(p2, r2, rsd_i, b2, g2, be2, gids, wspec_t, bspec_v,
      Wg, bgate, Wf1, bf1, gf1, bef1, Wf2, bf2, gf2, bef2, Wo, bo)


def kernel(node_feats, edge_index, graph_ids, W1, b1, Wr1, br1, g1, be1,
           W2, b2, Wr2, br2, g2, be2, Wspec, bspec, Wsh, bsh, Wg, bgate,
           Wf1, bf1, gf1, bef1, Wf2, bf2, gf2, bef2, Wo, bo):
    n = node_feats.shape[0]
    e = edge_index.shape[1]
    n_tasks = Wspec.shape[0]

    src = edge_index[0].astype(jnp.int32)
    dst = edge_index[1].astype(jnp.int32)
    nw = _NC * _NS
    src2 = src.reshape(nw, e // nw)
    dst2 = dst.reshape(nw, e // nw)

    src3 = src.reshape(_NC, _NS, e // nw)
    dst3 = dst.reshape(_NC, _NS, e // nw)

    quantum = 80 * _NS * 2
    n_pad = ((n + quantum - 1) // quantum) * quantum
    hist_o, hist_i = _sc_degree_hist(src2, dst2, n)
    xw, r1 = _tc_mats(node_feats, W1, Wr1, br1)
    h1, rsd_o, rsd_i = _tc_scale(xw, hist_o, hist_i)
    p1 = _sc_edge_aggregate(h1, src3, dst3, n_pad, kk=40, nb=5)
    h2, h1bn = _tc_mid_a(p1, r1, rsd_i, rsd_o, b1, g1, be1, W2)
    r2 = _tc_mid_b(h1bn, Wr2, br2)
    p2 = _sc_edge_aggregate(h2, src3, dst3, n_pad, kk=80, nb=5)

    wspec_t = jnp.transpose(Wspec[:, :, 0])
    bspec_v = bspec[:, 0]
    return _tc_final(p2, r2, rsd_i, b2, g2, be2, graph_ids.astype(jnp.int32),
                     wspec_t, bspec_v, Wg, bgate, Wf1, bf1, gf1, bef1,
                     Wf2, bf2, gf2, bef2, Wo, bo, _N_GRAPHS, n_tasks)


# speedup vs baseline: 11.5501x; 1.0177x over previous
import functools

import jax
import jax.numpy as jnp
from jax import lax
from jax.experimental import pallas as pl
from jax.experimental.pallas import tpu as pltpu
from jax.experimental.pallas import tpu_sc as plsc

_F32 = jnp.float32
_NC = 2
_NS = 16
_N_GRAPHS = 256
_NB = 5


def _mm(a, b):
    return lax.dot_general(
        a, b, (((a.ndim - 1,), (0,)), ((), ())),
        precision=lax.Precision.HIGHEST, preferred_element_type=_F32)


def _bn(x, gamma, beta, eps=1e-5):
    mu = jnp.mean(x, axis=0)
    var = jnp.mean((x - mu[None, :]) ** 2, axis=0)
    return gamma[None, :] * (x - mu[None, :]) / jnp.sqrt(var + eps)[None, :] + beta[None, :]


def _sc_degree_hist(src2, dst2, n_nodes):
    nw, e_t = src2.shape
    mesh = plsc.VectorSubcoreMesh(core_axis_name="c", subcore_axis_name="s")

    @functools.partial(
        pl.kernel, mesh=mesh,
        out_type=(jax.ShapeDtypeStruct((nw, n_nodes), _F32),
                  jax.ShapeDtypeStruct((nw, n_nodes), _F32)),
        scratch_types=[pltpu.VMEM((e_t,), jnp.int32),
                       pltpu.VMEM((e_t,), jnp.int32),
                       pltpu.VMEM((n_nodes,), _F32),
                       pltpu.VMEM((n_nodes,), _F32)],
        compiler_params=pltpu.CompilerParams(needs_layout_passes=False),
    )
    def deg_kernel(src_hbm, dst_hbm, out_o, out_i, src_v, dst_v, ho_v, hi_v):
        c = lax.axis_index("c")
        s = lax.axis_index("s")
        wid = s * _NC + c
        pltpu.sync_copy(src_hbm.at[wid], src_v)
        pltpu.sync_copy(dst_hbm.at[wid], dst_v)
        zeros = jnp.zeros((16,), _F32)

        @pl.loop(0, n_nodes // 16)
        def _zero(i):
            ho_v[pl.ds(i * 16, 16)] = zeros
            hi_v[pl.ds(i * 16, 16)] = zeros

        ones = jnp.ones((16,), _F32)

        @pl.loop(0, e_t // 16)
        def _hist(i):
            plsc.addupdate_scatter(ho_v, [src_v[pl.ds(i * 16, 16)]], ones)
            plsc.addupdate_scatter(hi_v, [dst_v[pl.ds(i * 16, 16)]], ones)

        pltpu.sync_copy(ho_v, out_o.at[wid])
        pltpu.sync_copy(hi_v, out_i.at[wid])

    return deg_kernel(src2, dst2)


def _sc_edge_aggregate(h_nd, src3, dst3, n_pad, kk, nb):
    nc, ns, epc = src3.shape
    nch = epc // kk
    src4 = src3.reshape(nc, ns, nch, kk)
    dst4 = dst3.reshape(nc, ns, nch, kk)
    d = h_nd.shape[1]
    rpt = n_pad // ns
    zn = rpt // kk
    assert rpt % kk == 0 and nch % nb == 0
    mesh = plsc.VectorSubcoreMesh(core_axis_name="c", subcore_axis_name="s")

    @functools.partial(
        pl.kernel, mesh=mesh,
        out_type=jax.ShapeDtypeStruct((nc, n_pad, d), _F32),
        scratch_types=[pltpu.VMEM((nch, kk), jnp.int32),
                       pltpu.VMEM((nch, kk), jnp.int32)]
                      + [pltpu.VMEM((kk, d), _F32) for _ in range(nb)]
                      + [pltpu.VMEM_SHARED((n_pad, d), _F32)]
                      + [pltpu.SemaphoreType.DMA for _ in range(2 * nb)],
        compiler_params=pltpu.CompilerParams(needs_layout_passes=False,
                                             use_tc_tiling_on_sc=False),
    )
    def agg_kernel(h_hbm, src_hbm, dst_hbm, out_hbm, src_v, dst_v, *rest):
        gbufs = rest[:nb]
        agg_sh = rest[nb]
        sems = rest[nb + 1:nb + 1 + nb]
        ssems = rest[nb + 1 + nb:]
        c = lax.axis_index("c")
        s = lax.axis_index("s")
        pltpu.sync_copy(src_hbm.at[c, s], src_v)
        pltpu.sync_copy(dst_hbm.at[c, s], dst_v)
        zeros = jnp.zeros((16,), _F32)

        @pl.loop(0, kk)
        def _zero(i):
            @pl.loop(0, d // 16)
            def _zero_row(j):
                gbufs[0][i, pl.ds(j * 16, 16)] = zeros

        @pl.loop(0, zn)
        def _zinit(t):
            pltpu.sync_copy(gbufs[0], agg_sh.at[pl.ds(s * rpt + t * kk, kk)])

        plsc.subcore_barrier()

        @pl.loop(0, nch // nb)
        def _rounds(g):
            j0 = g * nb
            cps = [pltpu.async_copy(h_hbm.at[src_v.at[j0 + b]], gbufs[b],
                                    sems[b]) for b in range(nb)]
            scs = []
            for b in range(nb):
                cps[b].wait()
                scs.append(pltpu.async_copy(gbufs[b],
                                            agg_sh.at[dst_v.at[j0 + b]],
                                            ssems[b], add=True))
            for b in range(nb):
                scs[b].wait()

        plsc.subcore_barrier()
        pltpu.sync_copy(agg_sh.at[pl.ds(s * rpt, rpt)],
                        out_hbm.at[c, pl.ds(s * rpt, rpt)])

    return agg_kernel(h_nd, src4, dst4)


def _tc_mats(x, W1, Wr1, br1):
    n, d_in = x.shape
    d_hid = W1.shape[1]

    def body(x_ref, w1_ref, wr1_ref, br1_ref, xw_out, r_out):
        xv = x_ref[...]
        xw_out[...] = _mm(xv, w1_ref[...])
        r_out[...] = jax.nn.relu(_mm(xv, wr1_ref[...]) + br1_ref[...][None, :])

    return pl.pallas_call(
        body,
        out_shape=(jax.ShapeDtypeStruct((n, d_hid), _F32),
                   jax.ShapeDtypeStruct((n, d_hid), _F32)),
    )(x, W1, Wr1, br1)


def _tc_scale(xw, hist_o, hist_i):
    n, d_hid = xw.shape

    def body(xw_ref, ho_ref, hi_ref, h_out, ro_out, ri_out):
        dego = jnp.maximum(jnp.sum(ho_ref[...], axis=0), 1.0)
        degi = jnp.maximum(jnp.sum(hi_ref[...], axis=0), 1.0)
        rsd_o = lax.rsqrt(dego)
        h_out[...] = xw_ref[...] * rsd_o[:, None]
        ro_out[...] = rsd_o
        ri_out[...] = lax.rsqrt(degi)

    return pl.pallas_call(
        body,
        out_shape=(jax.ShapeDtypeStruct((n, d_hid), _F32),
                   jax.ShapeDtypeStruct((n,), _F32),
                   jax.ShapeDtypeStruct((n,), _F32)),
    )(xw, hist_o, hist_i)


def _tc_mid_a(p, r1, rsd_i, rsd_o, b1, g1, be1, W2):
    n, d_hid = r1.shape
    d_out = W2.shape[1]

    def body(p_ref, r1_ref, ri_ref, ro_ref, b1_ref, g1_ref, be1_ref,
             w2_ref, h2_out, h1_out):
        agg = p_ref[0][:n] + p_ref[1][:n]
        pre = jax.nn.relu(agg * ri_ref[...][:, None] + b1_ref[...][None, :]) + r1_ref[...]
        h1 = _bn(pre, g1_ref[...], be1_ref[...])
        h2_out[...] = _mm(h1, w2_ref[...]) * ro_ref[...][:, None]
        h1_out[...] = h1

    return pl.pallas_call(
        body,
        out_shape=(jax.ShapeDtypeStruct((n, d_out), _F32),
                   jax.ShapeDtypeStruct((n, d_hid), _F32)),
    )(p, r1, rsd_i, rsd_o, b1, g1, be1, W2)


def _tc_mid_b(h1, Wr2, br2):
    n = h1.shape[0]
    d_out = Wr2.shape[1]

    def body(h1_ref, wr2_ref, br2_ref, r2_out):
        r2_out[...] = jax.nn.relu(_mm(h1_ref[...], wr2_ref[...]) + br2_ref[...][None, :])

    return pl.pallas_call(
        body,
        out_shape=jax.ShapeDtypeStruct((n, d_out), _F32),
    )(h1, Wr2, br2)


def _tc_final(p2, r2, rsd_i, b2, g2, be2, gids, wspec_t, bspec_v,
              Wg, bgate, Wf1, bf1, gf1, bef1, Wf2, bf2, gf2, bef2, Wo, bo,
              n_graphs, n_tasks):
    n, d_out = r2.shape

    def body(p_ref, r2_ref, ri_ref, b2_ref, g2_ref, be2_ref, gid_ref,
             wspec_ref, bspec_ref, wg_ref, bgate_ref, wf1_ref, bf1_ref,
             gf1_ref, bef1_ref, wf2_ref, bf2_ref, gf2_ref, bef2_ref,
             wo_ref, bo_ref, out_ref):
        agg = p_ref[0][:n] + p_ref[1][:n]
        pre = jax.nn.relu(agg * ri_ref[...][:, None] + b2_ref[...][None, :]) + r2_ref[...]
        h2 = _bn(pre, g2_ref[...], be2_ref[...])

        w = jax.nn.sigmoid(_mm(h2, wspec_ref[...]) + bspec_ref[...][None, :])
        ids = gid_ref[...]
        onehot = (ids[:, None] ==
                  lax.broadcasted_iota(jnp.int32, (n, n_graphs), 1)).astype(_F32)
        xcat = jnp.concatenate(
            [h2 * w[:, i][:, None] for i in range(n_tasks)] + [h2], axis=1)
        seg = lax.dot_general(onehot, xcat, (((0,), (0,)), ((), ())),
                              precision=lax.Precision.HIGHEST,
                              preferred_element_type=_F32)
        counts = jnp.maximum(jnp.sum(onehot, axis=0), 1.0)
        feats = [seg[:, i * d_out:(i + 1) * d_out] for i in range(n_tasks)]
        hg = seg[:, n_tasks * d_out:(n_tasks + 1) * d_out] / counts[:, None]
        prim = feats[n_tasks - 1]

        gc = jnp.zeros((n_graphs, d_out), _F32)
        for i in range(n_tasks - 1):
            logits = _mm(hg, wg_ref[i]) + bgate_ref[i][None, :]
            gate = jax.nn.softmax(logits, axis=-1)
            gc = gc + feats[i] * gate[:, 0][:, None] + prim * gate[:, 1][:, None]

        combine2 = [feats[0], gc, feats[1], feats[2], feats[3]]
        preds = []
        for i in range(n_tasks):
            a = jax.nn.relu(_mm(combine2[i], wf1_ref[i]) + bf1_ref[i][None, :])
            a = _bn(a, gf1_ref[i], bef1_ref[i])
            a = jax.nn.relu(_mm(a, wf2_ref[i]) + bf2_ref[i][None, :])
            a = _bn(a, gf2_ref[i], bef2_ref[i])
            preds.append(_mm(a, wo_ref[i]) + bo_ref[i][None, :])
        out_ref[...] = jnp.concatenate(preds, axis=1)

    return pl.pallas_call(
        body,
        out_shape=jax.ShapeDtypeStruct((n_graphs, n_tasks), _F32),
    )(p2, r2, rsd_i, b2, g2, be2, gids, wspec_t, bspec_v,
      Wg, bgate, Wf1, bf1, gf1, bef1, Wf2, bf2, gf2, bef2, Wo, bo)


def kernel(node_feats, edge_index, graph_ids, W1, b1, Wr1, br1, g1, be1,
           W2, b2, Wr2, br2, g2, be2, Wspec, bspec, Wsh, bsh, Wg, bgate,
           Wf1, bf1, gf1, bef1, Wf2, bf2, gf2, bef2, Wo, bo):
    n = node_feats.shape[0]
    e = edge_index.shape[1]
    n_tasks = Wspec.shape[0]

    src = edge_index[0].astype(jnp.int32)
    dst = edge_index[1].astype(jnp.int32)
    nw = _NC * _NS
    src2 = src.reshape(nw, e // nw)
    dst2 = dst.reshape(nw, e // nw)

    src3 = src.reshape(_NC, _NS, e // nw)
    dst3 = dst.reshape(_NC, _NS, e // nw)

    quantum = 80 * _NS * 2
    n_pad = ((n + quantum - 1) // quantum) * quantum
    hist_o, hist_i = _sc_degree_hist(src2, dst2, n)
    xw, r1 = _tc_mats(node_feats, W1, Wr1, br1)
    h1, rsd_o, rsd_i = _tc_scale(xw, hist_o, hist_i)
    p1 = _sc_edge_aggregate(h1, src3, dst3, n_pad, kk=40, nb=5)
    h2, h1bn = _tc_mid_a(p1, r1, rsd_i, rsd_o, b1, g1, be1, W2)
    r2 = _tc_mid_b(h1bn, Wr2, br2)
    p2 = _sc_edge_aggregate(h2, src3, dst3, n_pad, kk=40, nb=10)

    wspec_t = jnp.transpose(Wspec[:, :, 0])
    bspec_v = bspec[:, 0]
    return _tc_final(p2, r2, rsd_i, b2, g2, be2, graph_ids.astype(jnp.int32),
                     wspec_t, bspec_v, Wg, bgate, Wf1, bf1, gf1, bef1,
                     Wf2, bf2, gf2, bef2, Wo, bo, _N_GRAPHS, n_tasks)
